# baseline (speedup 1.0000x reference)
import jax
import jax.numpy as jnp
from jax import lax
from jax.experimental import pallas as pl
from jax.experimental.pallas import tpu as pltpu
from jax.experimental.pallas import tpu_sc as plsc

_B = 16
_N = 8400
_G = _B * _N
_CHUNK = 4224
_LAST = _G - _CHUNK
_INV = 1.0 / 80.0
_NP = 8448
_SPLIT0 = 2048


def _roi_body(yolo_hbm, out_hbm, idx_v, in_v, out_v, sem):
    cid = lax.axis_index("c")
    sid = lax.axis_index("s")
    wid = sid * 2 + cid
    base = jnp.where(wid == 31, _LAST, wid * _CHUNK)
    b_lo = base // _N
    b_hi = jnp.minimum(b_lo + 1, _B - 1)
    lane = jnp.arange(16, dtype=jnp.int32)
    idx_v[...] = (lane & 3) * _B + jnp.where(lane >= 4, b_hi, b_lo)
    gather = pltpu.async_copy(yolo_hbm.at[idx_v.at[pl.ds(0, 8)]], in_v, sem)

    split = (b_lo + 1) * _N
    f_lo = b_lo.astype(jnp.float32)
    f_hi = b_hi.astype(jnp.float32)

    def fill(i, carry):
        for k in range(8):
            off = i * 128 + k * 16
            g = base + off
            bf = jnp.broadcast_to(jnp.where(g >= split, f_hi, f_lo), (16,))
            out_v[0, pl.ds(off, 16)] = bf
        return carry

    lax.fori_loop(0, _CHUNK // 128, fill, 0)
    gather.wait()

    def step(i, carry):
        for k in range(8):
            off = i * 128 + k * 16
            g = base + off
            in_hi = g >= split
            rb = jnp.where(in_hi, 4, 0)
            n = g - jnp.where(in_hi, split, b_lo * _N)
            cx = in_v[rb + 0, pl.ds(n, 16)]
            cy = in_v[rb + 1, pl.ds(n, 16)]
            w = in_v[rb + 2, pl.ds(n, 16)]
            h = in_v[rb + 3, pl.ds(n, 16)]
            hw = w * 0.5
            hh = h * 0.5
            x1 = (cx - hw) * _INV
            x2 = (cx + hw) * _INV
            y1 = (cy - hh) * _INV
            y2 = (cy + hh) * _INV
            out_v[1, pl.ds(off, 16)] = jnp.clip(jnp.minimum(x1, x2), 0.0, 1.0)
            out_v[2, pl.ds(off, 16)] = jnp.clip(jnp.minimum(y1, y2), 0.0, 1.0)
            out_v[3, pl.ds(off, 16)] = jnp.clip(jnp.maximum(x1, x2), 0.0, 1.0)
            out_v[4, pl.ds(off, 16)] = jnp.clip(jnp.maximum(y1, y2), 0.0, 1.0)
        return carry

    lax.fori_loop(0, _SPLIT0 // 128, step, 0)
    d0 = pltpu.async_copy(
        out_v.at[:, pl.ds(0, _SPLIT0)],
        out_hbm.at[:, pl.ds(base, _SPLIT0)],
        sem,
    )
    lax.fori_loop(_SPLIT0 // 128, _CHUNK // 128, step, 0)
    d1 = pltpu.async_copy(
        out_v.at[:, pl.ds(_SPLIT0, _CHUNK - _SPLIT0)],
        out_hbm.at[:, pl.ds(base + _SPLIT0, _CHUNK - _SPLIT0)],
        sem,
    )
    d0.wait()
    d1.wait()


def kernel(yolo_output, input_images_or_features):
    del input_images_or_features
    boxes = jnp.pad(yolo_output[:, :4, :], ((0, 0), (0, 0), (0, _NP - _N)))
    rows = boxes.transpose(1, 0, 2).reshape(4 * _B, _NP)
    mesh = plsc.VectorSubcoreMesh(core_axis_name="c", subcore_axis_name="s")
    run = pl.kernel(
        _roi_body,
        out_type=jax.ShapeDtypeStruct((5, _G), jnp.float32),
        mesh=mesh,
        scratch_types=[
            pltpu.VMEM((16,), jnp.int32),
            pltpu.VMEM((8, _NP), jnp.float32),
            pltpu.VMEM((5, _CHUNK), jnp.float32),
            pltpu.SemaphoreType.DMA,
        ],
        compiler_params=pltpu.CompilerParams(
            needs_layout_passes=False,
            skip_device_barrier=True,
        ),
    )
    planes = run(rows)
    return planes.T


# speedup vs baseline: 1.0193x; 1.0193x over previous
import jax
import jax.numpy as jnp
from jax import lax
from jax.experimental import pallas as pl
from jax.experimental.pallas import tpu as pltpu
from jax.experimental.pallas import tpu_sc as plsc

_B = 16
_N = 8400
_G = _B * _N
_CHUNK = 4224
_NV = _CHUNK // 16
_LAST = _G - _CHUNK
_INV = 1.0 / 80.0
_NP = 8448


def _roi_body(yolo_hbm, out_hbm, idx_v, in_v, out_v, sem):
    cid = lax.axis_index("c")
    sid = lax.axis_index("s")
    wid = sid * 2 + cid
    base = jnp.where(wid == 31, _LAST, wid * _CHUNK)
    b_lo = base // _N
    b_hi = jnp.minimum(b_lo + 1, _B - 1)
    lane = jnp.arange(16, dtype=jnp.int32)
    idx_v[...] = (lane & 3) * _B + jnp.where(lane >= 4, b_hi, b_lo)
    pltpu.async_copy(yolo_hbm.at[idx_v.at[pl.ds(0, 8)]], in_v, sem).wait()

    split = (b_lo + 1) * _N
    f_lo = b_lo.astype(jnp.float32)
    f_hi = b_hi.astype(jnp.float32)

    def step(i, carry):
        off = i * 16
        g = base + off
        in_hi = g >= split
        rb = jnp.where(in_hi, 4, 0)
        n = g - jnp.where(in_hi, split, b_lo * _N)
        cx = in_v[rb + 0, pl.ds(n, 16)]
        cy = in_v[rb + 1, pl.ds(n, 16)]
        w = in_v[rb + 2, pl.ds(n, 16)]
        h = in_v[rb + 3, pl.ds(n, 16)]
        hw = w * 0.5
        hh = h * 0.5
        x1 = (cx - hw) * _INV
        x2 = (cx + hw) * _INV
        y1 = (cy - hh) * _INV
        y2 = (cy + hh) * _INV
        bf = jnp.broadcast_to(jnp.where(in_hi, f_hi, f_lo), (16,))
        out_v[0, pl.ds(off, 16)] = bf
        out_v[1, pl.ds(off, 16)] = jnp.clip(jnp.minimum(x1, x2), 0.0, 1.0)
        out_v[2, pl.ds(off, 16)] = jnp.clip(jnp.minimum(y1, y2), 0.0, 1.0)
        out_v[3, pl.ds(off, 16)] = jnp.clip(jnp.maximum(x1, x2), 0.0, 1.0)
        out_v[4, pl.ds(off, 16)] = jnp.clip(jnp.maximum(y1, y2), 0.0, 1.0)
        return carry

    lax.fori_loop(0, _NV, step, 0)
    pltpu.sync_copy(out_v, out_hbm.at[:, pl.ds(base, _CHUNK)])


def kernel(yolo_output, input_images_or_features):
    del input_images_or_features
    boxes = jnp.pad(yolo_output[:, :4, :], ((0, 0), (0, 0), (0, _NP - _N)))
    rows = boxes.transpose(1, 0, 2).reshape(4 * _B, _NP)
    mesh = plsc.VectorSubcoreMesh(core_axis_name="c", subcore_axis_name="s")
    run = pl.kernel(
        _roi_body,
        out_type=jax.ShapeDtypeStruct((5, _G), jnp.float32),
        mesh=mesh,
        scratch_types=[
            pltpu.VMEM((16,), jnp.int32),
            pltpu.VMEM((8, _NP), jnp.float32),
            pltpu.VMEM((5, _CHUNK), jnp.float32),
            pltpu.SemaphoreType.DMA,
        ],
        compiler_params=pltpu.CompilerParams(
            needs_layout_passes=False,
            skip_device_barrier=True,
        ),
    )
    planes = run(rows)
    return planes.T
